# baseline (speedup 1.0000x reference)
import functools

import jax
import jax.numpy as jnp
from jax import lax
from jax.experimental import pallas as pl
from jax.experimental.pallas import tpu as pltpu
from jax.experimental.pallas import tpu_sc as plsc

NN = 10000
NP = 10240
EE = 320000
DD = 128
HH = 128
H2 = 64
CC = 10
BN_EPS = 1e-5
W1 = 80

NSC = 2
NTILES = 16
CH = 128
NCHUNK = 160
NB = 5
EP = NTILES * NCHUNK * CH
RPT = NP // NTILES

BLK = 1000
GRID = NN // BLK


def _make_segsum(width):
  mesh = plsc.VectorSubcoreMesh(core_axis_name="c", subcore_axis_name="s")
  L = NB - 2

  @functools.partial(
      pl.kernel,
      out_type=jax.ShapeDtypeStruct((NSC, NP, width), jnp.float32),
      mesh=mesh,
      scratch_types=(
          pltpu.VMEM((NCHUNK, CH), jnp.int32),
          pltpu.VMEM((NB, CH), jnp.int32),
          pltpu.VMEM((NB, CH), jnp.int32),
          pltpu.VMEM((NB, CH, width), jnp.float32),
          pltpu.VMEM_SHARED((NP, width), jnp.float32),
          pltpu.SemaphoreType.DMA,
          pltpu.SemaphoreType.DMA,
      ),
      compiler_params=pltpu.CompilerParams(use_tc_tiling_on_sc=False))
  def body(ta, tb, pidx_hbm, zrow, out, pidx, st_s, st_d, rows, acc,
           gsem, ssem):
    c = lax.axis_index("c")
    s = lax.axis_index("s")

    pltpu.sync_copy(pidx_hbm.at[s], pidx)
    pltpu.sync_copy(zrow, acc.at[pl.ds(s * RPT, RPT)])
    plsc.subcore_barrier()

    def unpack(j, slot):
      for k in range(CH // 16):
        pk = pidx[j, pl.ds(k * 16, 16)]
        st_s[slot, pl.ds(k * 16, 16)] = pk & 16383
        st_d[slot, pl.ds(k * 16, 16)] = jnp.right_shift(pk, 14)

    def run(table):
      def gather(j_slot):
        return pltpu.async_copy(table.at[st_s.at[j_slot]], rows.at[j_slot],
                                gsem)

      def gather_wait(j_slot):
        pltpu.make_async_copy(table.at[st_s.at[j_slot]], rows.at[j_slot],
                              gsem).wait()

      def scatter(j_slot):
        return pltpu.async_copy(rows.at[j_slot], acc.at[st_d.at[j_slot]],
                                ssem, add=True)

      def scatter_wait(j_slot):
        pltpu.make_async_copy(rows.at[j_slot], acc.at[st_d.at[j_slot]],
                              ssem).wait()

      for j0 in range(L):
        unpack(j0, j0)
        gather(j0)

      def group(g, carry):
        for b in range(NB):
          j = g * NB + b
          slot_next = (b + L) % NB
          if b >= 2:
            scatter_wait((b - 2) % NB)
          else:
            @pl.when(g > 0)
            def _():
              scatter_wait((b - 2) % NB)
          jc = jnp.minimum(j + L, NCHUNK - 1)
          unpack(jc, slot_next)
          gather(slot_next)
          gather_wait(b)
          scatter(b)
        return carry

      lax.fori_loop(0, NCHUNK // NB, group, 0)

      scatter_wait((NCHUNK - 2) % NB)
      scatter_wait((NCHUNK - 1) % NB)
      for e in range(L):
        gather_wait(e)

    @pl.when(c == 0)
    def _():
      run(ta)

    @pl.when(c == 1)
    def _():
      run(tb)

    plsc.subcore_barrier()

    pltpu.sync_copy(acc.at[pl.ds(s * RPT, RPT)],
                    out.at[c, pl.ds(s * RPT, RPT)])

  return body


_segsum80 = _make_segsum(W1)
_segsum32 = _make_segsum(H2 // 2)


def _mm(a, b_t):
  return lax.dot_general(a, b_t, (((1,), (1,)), ((), ())),
                         preferred_element_type=jnp.float32)


def _tc1_body(x_ref, wl_ref, wr_ref, qa_ref, qb_ref, r_ref):
  xb = x_ref[...]
  q = _mm(xb, wl_ref[...])
  r_ref[...] = _mm(xb, wr_ref[...])
  onescol = jnp.concatenate(
      [jnp.ones((BLK, 1), jnp.float32), jnp.zeros((BLK, 15), jnp.float32)],
      axis=1)
  qa_ref[...] = jnp.concatenate([q[:, :H2], onescol], axis=1)
  qb_ref[...] = jnp.concatenate([q[:, H2:], jnp.zeros((BLK, 16), jnp.float32)],
                                axis=1)


def _tc1(x, w1l, w1r):
  return pl.pallas_call(
      _tc1_body,
      grid=(GRID,),
      in_specs=[
          pl.BlockSpec((BLK, DD), lambda i: (i, 0)),
          pl.BlockSpec((HH, DD), lambda i: (0, 0)),
          pl.BlockSpec((HH, DD), lambda i: (0, 0)),
      ],
      out_specs=[
          pl.BlockSpec((BLK, W1), lambda i: (i, 0)),
          pl.BlockSpec((BLK, W1), lambda i: (i, 0)),
          pl.BlockSpec((BLK, HH), lambda i: (i, 0)),
      ],
      out_shape=[
          jax.ShapeDtypeStruct((NN, W1), jnp.float32),
          jax.ShapeDtypeStruct((NN, W1), jnp.float32),
          jax.ShapeDtypeStruct((NN, HH), jnp.float32),
      ],
  )(x, w1l, w1r)


def _tc2_body(agg_ref, r1_ref, b1_ref, g1_ref, be1_ref,
              w2l_ref, w2r_ref, pa_ref, pb_ref, r2_ref, scl_ref):
  scale = 1.0 / jnp.maximum(agg_ref[0, :, H2:H2 + 1], 1.0)
  agg = jnp.concatenate([agg_ref[0, :, :H2], agg_ref[1, :, :H2]],
                        axis=1) * scale
  h = agg + b1_ref[...] + r1_ref[...]
  h = h * (1.0 / jnp.sqrt(1.0 + BN_EPS)) * g1_ref[...] + be1_ref[...]
  h = jnp.maximum(h, 0.0)
  p = _mm(h, w2l_ref[...])
  r2_ref[...] = _mm(h, w2r_ref[...])
  pa_ref[...] = p[:, :H2 // 2]
  pb_ref[...] = p[:, H2 // 2:]
  scl_ref[...] = jnp.broadcast_to(scale, (BLK, 16))


def _tc2(agg1, r1, b1, g1, be1, w2l, w2r):
  return pl.pallas_call(
      _tc2_body,
      grid=(GRID,),
      in_specs=[
          pl.BlockSpec((NSC, BLK, W1), lambda i: (0, i, 0)),
          pl.BlockSpec((BLK, HH), lambda i: (i, 0)),
          pl.BlockSpec((1, HH), lambda i: (0, 0)),
          pl.BlockSpec((1, HH), lambda i: (0, 0)),
          pl.BlockSpec((1, HH), lambda i: (0, 0)),
          pl.BlockSpec((H2, HH), lambda i: (0, 0)),
          pl.BlockSpec((H2, HH), lambda i: (0, 0)),
      ],
      out_specs=[
          pl.BlockSpec((BLK, H2 // 2), lambda i: (i, 0)),
          pl.BlockSpec((BLK, H2 // 2), lambda i: (i, 0)),
          pl.BlockSpec((BLK, H2), lambda i: (i, 0)),
          pl.BlockSpec((BLK, 16), lambda i: (i, 0)),
      ],
      out_shape=[
          jax.ShapeDtypeStruct((NN, H2 // 2), jnp.float32),
          jax.ShapeDtypeStruct((NN, H2 // 2), jnp.float32),
          jax.ShapeDtypeStruct((NN, H2), jnp.float32),
          jax.ShapeDtypeStruct((NN, 16), jnp.float32),
      ],
  )(agg1, r1, b1, g1, be1, w2l, w2r)


def _tc3_body(agg_ref, scl_ref, r2_ref, b2_ref, g2_ref, be2_ref,
              wh_ref, bh_ref, o_ref):
  scale = scl_ref[:, 0:1]
  agg = jnp.concatenate([agg_ref[0], agg_ref[1]], axis=1) * scale
  h = agg + b2_ref[...] + r2_ref[...]
  h = h * (1.0 / jnp.sqrt(1.0 + BN_EPS)) * g2_ref[...] + be2_ref[...]
  h = jnp.maximum(h, 0.0)
  o_ref[...] = lax.dot_general(h, wh_ref[...], (((1,), (0,)), ((), ())),
                               preferred_element_type=jnp.float32) + bh_ref[...]


def _tc3(agg2, scl, r2, b2, g2, be2, whp, bhp):
  return pl.pallas_call(
      _tc3_body,
      grid=(GRID,),
      in_specs=[
          pl.BlockSpec((NSC, BLK, H2 // 2), lambda i: (0, i, 0)),
          pl.BlockSpec((BLK, 16), lambda i: (i, 0)),
          pl.BlockSpec((BLK, H2), lambda i: (i, 0)),
          pl.BlockSpec((1, H2), lambda i: (0, 0)),
          pl.BlockSpec((1, H2), lambda i: (0, 0)),
          pl.BlockSpec((1, H2), lambda i: (0, 0)),
          pl.BlockSpec((H2, 128), lambda i: (0, 0)),
          pl.BlockSpec((1, 128), lambda i: (0, 0)),
      ],
      out_specs=pl.BlockSpec((BLK, 128), lambda i: (i, 0)),
      out_shape=jax.ShapeDtypeStruct((NN, 128), jnp.float32),
  )(agg2, scl, r2, b2, g2, be2, whp, bhp)


def kernel(x, edge_index, W1_l, b1_l, W1_r, g1, be1,
           W2_l, b2_l, W2_r, g2, be2, Wh, bh):
  pad = EP - EE
  src_p = jnp.concatenate([edge_index[0], jnp.zeros((pad,), jnp.int32)])
  dst_p = jnp.concatenate([edge_index[1], jnp.full((pad,), NN, jnp.int32)])
  pidx = (src_p | (dst_p << 14)).reshape(NTILES, NCHUNK, CH)

  zrow80 = jnp.zeros((RPT, W1), jnp.float32)
  zrow32 = jnp.zeros((RPT, H2 // 2), jnp.float32)

  qa, qb, r1 = _tc1(x, W1_l, W1_r)
  agg1 = _segsum80(qa, qb, pidx, zrow80)

  pa, pb, r2, scl = _tc2(agg1, r1, b1_l.reshape(1, HH), g1.reshape(1, HH),
                         be1.reshape(1, HH), W2_l, W2_r)
  agg2 = _segsum32(pa, pb, pidx, zrow32)

  whp = jnp.zeros((H2, 128), jnp.float32).at[:, :CC].set(Wh.T)
  bhp = jnp.zeros((1, 128), jnp.float32).at[0, :CC].set(bh)
  out = _tc3(agg2, scl, r2, b2_l.reshape(1, H2), g2.reshape(1, H2),
             be2.reshape(1, H2), whp, bhp)
  return out[:, :CC]


# speedup vs baseline: 1.2421x; 1.2421x over previous
import functools

import jax
import jax.numpy as jnp
from jax import lax
from jax.experimental import pallas as pl
from jax.experimental.pallas import tpu as pltpu
from jax.experimental.pallas import tpu_sc as plsc

NN = 10000
NP = 10240
EE = 320000
ER = 2500
ERP = 2560
DD = 128
HH = 128
H2 = 64
CC = 10
BN_EPS = 1e-5

NSC = 2
NTILES = 16
CH = 128
NCHUNK = 160
RPT = NP // NTILES

BLK = 1000
GRID = NN // BLK
EBLK = ERP // GRID


def _make_segsum(width, with_cnt, NB):
  mesh = plsc.VectorSubcoreMesh(core_axis_name="c", subcore_axis_name="s")

  out_type = [jax.ShapeDtypeStruct((NSC, NP, width), jnp.float32)]
  scratch = [
      pltpu.VMEM((NCHUNK, CH), jnp.int32),
      pltpu.VMEM((NCHUNK, CH), jnp.int32),
      pltpu.VMEM((NB, CH, width), jnp.float32),
      pltpu.VMEM_SHARED((NP, width), jnp.float32),
      pltpu.SemaphoreType.DMA,
      pltpu.SemaphoreType.DMA,
  ]
  if with_cnt:
    out_type.append(jax.ShapeDtypeStruct((NSC, NP, 16), jnp.float32))
    scratch += [
        pltpu.VMEM((CH, 16), jnp.float32),
        pltpu.VMEM_SHARED((NP, 16), jnp.float32),
        pltpu.SemaphoreType.DMA,
    ]

  def body(*refs):
    if with_cnt:
      (ta, tb, src_r, dst_r, zrow, zcnt, ones16,
       out, cnt_out, idx_s, idx_d, rows, acc, gsem, ssem,
       onesb, cacc, osem) = refs
    else:
      (ta, tb, src_r, dst_r, zrow,
       out, idx_s, idx_d, rows, acc, gsem, ssem) = refs
    c = lax.axis_index("c")
    s = lax.axis_index("s")

    pltpu.sync_copy(src_r.at[s], idx_s)
    pltpu.sync_copy(dst_r.at[s], idx_d)
    pltpu.sync_copy(zrow, acc.at[pl.ds(s * RPT, RPT)])
    if with_cnt:
      pltpu.sync_copy(ones16, onesb)
      pltpu.sync_copy(zcnt, cacc.at[pl.ds(s * RPT, RPT)])

    plsc.subcore_barrier()

    def run(table, parity):
      def group(g, carry):
        base = g * NB
        gds = [pltpu.async_copy(table.at[idx_s.at[base + b]], rows.at[b], gsem)
               for b in range(NB)]
        sds = []
        for b in range(NB):
          gds[b].wait()
          sds.append(pltpu.async_copy(rows.at[b], acc.at[idx_d.at[base + b]],
                                      ssem, add=True))
          if with_cnt and b % 2 == parity:
            sds.append(pltpu.async_copy(onesb, cacc.at[idx_d.at[base + b]],
                                        osem, add=True))
        for d in sds:
          d.wait()
        return carry
      lax.fori_loop(0, NCHUNK // NB, group, 0)

    @pl.when(c == 0)
    def _():
      run(ta, 0)

    @pl.when(c == 1)
    def _():
      run(tb, 1)

    plsc.subcore_barrier()

    pltpu.sync_copy(acc.at[pl.ds(s * RPT, RPT)],
                    out.at[c, pl.ds(s * RPT, RPT)])
    if with_cnt:
      pltpu.sync_copy(cacc.at[pl.ds(s * RPT, RPT)],
                      cnt_out.at[c, pl.ds(s * RPT, RPT)])

  return functools.partial(
      pl.kernel, out_type=tuple(out_type), mesh=mesh,
      scratch_types=tuple(scratch),
      compiler_params=pltpu.CompilerParams(use_tc_tiling_on_sc=False))(body)


_segsum64_cnt = _make_segsum(H2, True, 4)
_segsum32 = _make_segsum(H2 // 2, False, 8)


def _mm(a, b_t):
  return lax.dot_general(a, b_t, (((1,), (1,)), ((), ())),
                         preferred_element_type=jnp.float32)


def _tc1_body(x_ref, wl_ref, wr_ref, e_ref,
              qa_ref, qb_ref, r_ref, src_ref, dst_ref):
  xb = x_ref[...]
  q = _mm(xb, wl_ref[...])
  r_ref[...] = _mm(xb, wr_ref[...])
  qa_ref[...] = q[:, :H2]
  qb_ref[...] = q[:, H2:]
  i = pl.program_id(0)
  row = jax.lax.broadcasted_iota(jnp.int32, (EBLK, CH), 0) + i * EBLK
  valid = row < ER
  src_ref[...] = jnp.where(valid, e_ref[0], 0)
  dst_ref[...] = jnp.where(valid, e_ref[1], NN)


def _tc1(x, w1l, w1r, e3):
  return pl.pallas_call(
      _tc1_body,
      grid=(GRID,),
      in_specs=[
          pl.BlockSpec((BLK, DD), lambda i: (i, 0)),
          pl.BlockSpec((HH, DD), lambda i: (0, 0)),
          pl.BlockSpec((HH, DD), lambda i: (0, 0)),
          pl.BlockSpec((2, EBLK, CH), lambda i: (0, i, 0)),
      ],
      out_specs=[
          pl.BlockSpec((BLK, H2), lambda i: (i, 0)),
          pl.BlockSpec((BLK, H2), lambda i: (i, 0)),
          pl.BlockSpec((BLK, HH), lambda i: (i, 0)),
          pl.BlockSpec((EBLK, CH), lambda i: (i, 0)),
          pl.BlockSpec((EBLK, CH), lambda i: (i, 0)),
      ],
      out_shape=[
          jax.ShapeDtypeStruct((NN, H2), jnp.float32),
          jax.ShapeDtypeStruct((NN, H2), jnp.float32),
          jax.ShapeDtypeStruct((NN, HH), jnp.float32),
          jax.ShapeDtypeStruct((ERP, CH), jnp.int32),
          jax.ShapeDtypeStruct((ERP, CH), jnp.int32),
      ],
  )(x, w1l, w1r, e3)


def _tc2_body(agg_ref, cnt_ref, r1_ref, b1_ref, g1_ref, be1_ref,
              w2l_ref, w2r_ref, pa_ref, pb_ref, r2_ref, scl_ref):
  scale = 1.0 / jnp.maximum(cnt_ref[0, :, 0:1] + cnt_ref[1, :, 0:1], 1.0)
  agg = jnp.concatenate([agg_ref[0], agg_ref[1]], axis=1) * scale
  h = agg + b1_ref[...] + r1_ref[...]
  h = h * (1.0 / jnp.sqrt(1.0 + BN_EPS)) * g1_ref[...] + be1_ref[...]
  h = jnp.maximum(h, 0.0)
  p = _mm(h, w2l_ref[...])
  r2_ref[...] = _mm(h, w2r_ref[...])
  pa_ref[...] = p[:, :H2 // 2]
  pb_ref[...] = p[:, H2 // 2:]
  scl_ref[...] = jnp.broadcast_to(scale, (BLK, 16))


def _tc2(agg1, cnt, r1, b1, g1, be1, w2l, w2r):
  return pl.pallas_call(
      _tc2_body,
      grid=(GRID,),
      in_specs=[
          pl.BlockSpec((NSC, BLK, H2), lambda i: (0, i, 0)),
          pl.BlockSpec((NSC, BLK, 16), lambda i: (0, i, 0)),
          pl.BlockSpec((BLK, HH), lambda i: (i, 0)),
          pl.BlockSpec((1, HH), lambda i: (0, 0)),
          pl.BlockSpec((1, HH), lambda i: (0, 0)),
          pl.BlockSpec((1, HH), lambda i: (0, 0)),
          pl.BlockSpec((H2, HH), lambda i: (0, 0)),
          pl.BlockSpec((H2, HH), lambda i: (0, 0)),
      ],
      out_specs=[
          pl.BlockSpec((BLK, H2 // 2), lambda i: (i, 0)),
          pl.BlockSpec((BLK, H2 // 2), lambda i: (i, 0)),
          pl.BlockSpec((BLK, H2), lambda i: (i, 0)),
          pl.BlockSpec((BLK, 16), lambda i: (i, 0)),
      ],
      out_shape=[
          jax.ShapeDtypeStruct((NN, H2 // 2), jnp.float32),
          jax.ShapeDtypeStruct((NN, H2 // 2), jnp.float32),
          jax.ShapeDtypeStruct((NN, H2), jnp.float32),
          jax.ShapeDtypeStruct((NN, 16), jnp.float32),
      ],
  )(agg1, cnt, r1, b1, g1, be1, w2l, w2r)


def _tc3_body(agg_ref, scl_ref, r2_ref, b2_ref, g2_ref, be2_ref,
              wh_ref, bh_ref, o_ref):
  scale = scl_ref[:, 0:1]
  agg = jnp.concatenate([agg_ref[0], agg_ref[1]], axis=1) * scale
  h = agg + b2_ref[...] + r2_ref[...]
  h = h * (1.0 / jnp.sqrt(1.0 + BN_EPS)) * g2_ref[...] + be2_ref[...]
  h = jnp.maximum(h, 0.0)
  o_ref[...] = lax.dot_general(h, wh_ref[...], (((1,), (0,)), ((), ())),
                               preferred_element_type=jnp.float32) + bh_ref[...]


def _tc3(agg2, scl, r2, b2, g2, be2, whp, bhp):
  return pl.pallas_call(
      _tc3_body,
      grid=(GRID,),
      in_specs=[
          pl.BlockSpec((NSC, BLK, H2 // 2), lambda i: (0, i, 0)),
          pl.BlockSpec((BLK, 16), lambda i: (i, 0)),
          pl.BlockSpec((BLK, H2), lambda i: (i, 0)),
          pl.BlockSpec((1, H2), lambda i: (0, 0)),
          pl.BlockSpec((1, H2), lambda i: (0, 0)),
          pl.BlockSpec((1, H2), lambda i: (0, 0)),
          pl.BlockSpec((H2, 128), lambda i: (0, 0)),
          pl.BlockSpec((1, 128), lambda i: (0, 0)),
      ],
      out_specs=pl.BlockSpec((BLK, 128), lambda i: (i, 0)),
      out_shape=jax.ShapeDtypeStruct((NN, 128), jnp.float32),
  )(agg2, scl, r2, b2, g2, be2, whp, bhp)


def kernel(x, edge_index, W1_l, b1_l, W1_r, g1, be1,
           W2_l, b2_l, W2_r, g2, be2, Wh, bh):
  e3 = edge_index.reshape(2, ER, CH)

  zrow64 = jnp.zeros((RPT, H2), jnp.float32)
  zrow32 = jnp.zeros((RPT, H2 // 2), jnp.float32)
  zcnt = jnp.zeros((RPT, 16), jnp.float32)
  ones16 = jnp.zeros((CH, 16), jnp.float32).at[:, 0].set(1.0)

  qa, qb, r1, src_f, dst_f = _tc1(x, W1_l, W1_r, e3)
  src_r = src_f.reshape(NTILES, NCHUNK, CH)
  dst_r = dst_f.reshape(NTILES, NCHUNK, CH)
  agg1, cnt = _segsum64_cnt(qa, qb, src_r, dst_r, zrow64, zcnt, ones16)

  pa, pb, r2, scl = _tc2(agg1, cnt, r1, b1_l.reshape(1, HH), g1.reshape(1, HH),
                         be1.reshape(1, HH), W2_l, W2_r)
  agg2 = _segsum32(pa, pb, src_r, dst_r, zrow32)[0]

  whp = jnp.zeros((H2, 128), jnp.float32).at[:, :CC].set(Wh.T)
  bhp = jnp.zeros((1, 128), jnp.float32).at[0, :CC].set(bh)
  out = _tc3(agg2, scl, r2, b2_l.reshape(1, H2), g2.reshape(1, H2),
             be2.reshape(1, H2), whp, bhp)
  return out[:, :CC]


# speedup vs baseline: 1.3010x; 1.0474x over previous
import functools

import jax
import jax.numpy as jnp
from jax import lax
from jax.experimental import pallas as pl
from jax.experimental.pallas import tpu as pltpu
from jax.experimental.pallas import tpu_sc as plsc

NN = 10000
NP = 10240
EE = 320000
ER = 2500
ERP = 2560
DD = 128
HH = 128
H2 = 64
CC = 10
BN_EPS = 1e-5

NSC = 2
NTILES = 16
CH = 128
NCHUNK = 160
RPT = NP // NTILES

BLK = 1000
GRID = NN // BLK
EBLK = ERP // GRID


def _make_segsum(width, with_cnt, NB):
  mesh = plsc.VectorSubcoreMesh(core_axis_name="c", subcore_axis_name="s")

  out_type = [jax.ShapeDtypeStruct((NP, 128), jnp.float32)]
  scratch = [
      pltpu.VMEM((NCHUNK, CH), jnp.int32),
      pltpu.VMEM((NCHUNK, CH), jnp.int32),
      pltpu.VMEM((NB, CH, width), jnp.float32),
      pltpu.VMEM_SHARED((NP, width), jnp.float32),
      pltpu.SemaphoreType.DMA,
      pltpu.SemaphoreType.DMA,
  ]
  if with_cnt:
    out_type.append(jax.ShapeDtypeStruct((NSC, NP, 16), jnp.float32))
    scratch += [
        pltpu.VMEM((CH, 16), jnp.float32),
        pltpu.VMEM_SHARED((NP, 16), jnp.float32),
        pltpu.SemaphoreType.DMA,
    ]

  def body(*refs):
    if with_cnt:
      (ta, tb, src_r, dst_r, zrow, zcnt, ones16,
       out, cnt_out, idx_s, idx_d, rows, acc, gsem, ssem,
       onesb, cacc, osem) = refs
    else:
      (ta, tb, src_r, dst_r, zrow,
       out, idx_s, idx_d, rows, acc, gsem, ssem) = refs
    c = lax.axis_index("c")
    s = lax.axis_index("s")

    pltpu.sync_copy(src_r.at[s], idx_s)
    pltpu.sync_copy(dst_r.at[s], idx_d)
    pltpu.sync_copy(zrow, acc.at[pl.ds(s * RPT, RPT)])
    if with_cnt:
      pltpu.sync_copy(ones16, onesb)
      pltpu.sync_copy(zcnt, cacc.at[pl.ds(s * RPT, RPT)])

    plsc.subcore_barrier()

    def run(table, parity):
      def group(g, carry):
        base = g * NB
        gds = [pltpu.async_copy(table.at[idx_s.at[base + b]], rows.at[b], gsem)
               for b in range(NB)]
        sds = []
        for b in range(NB):
          gds[b].wait()
          sds.append(pltpu.async_copy(rows.at[b], acc.at[idx_d.at[base + b]],
                                      ssem, add=True))
          if with_cnt and b % 2 == parity:
            sds.append(pltpu.async_copy(onesb, cacc.at[idx_d.at[base + b]],
                                        osem, add=True))
        for d in sds:
          d.wait()
        return carry
      lax.fori_loop(0, NCHUNK // NB, group, 0)

    @pl.when(c == 0)
    def _():
      run(ta, 0)

    @pl.when(c == 1)
    def _():
      run(tb, 1)

    plsc.subcore_barrier()

    @pl.when(c == 0)
    def _():
      pltpu.sync_copy(acc.at[pl.ds(s * RPT, RPT)],
                      out.at[pl.ds(s * RPT, RPT), pl.ds(0, width)])

    @pl.when(c == 1)
    def _():
      pltpu.sync_copy(acc.at[pl.ds(s * RPT, RPT)],
                      out.at[pl.ds(s * RPT, RPT), pl.ds(width, width)])

    if with_cnt:
      pltpu.sync_copy(cacc.at[pl.ds(s * RPT, RPT)],
                      cnt_out.at[c, pl.ds(s * RPT, RPT)])

  return functools.partial(
      pl.kernel, out_type=tuple(out_type), mesh=mesh,
      scratch_types=tuple(scratch),
      compiler_params=pltpu.CompilerParams(use_tc_tiling_on_sc=False))(body)


_segsum64_cnt = _make_segsum(H2, True, 4)
_segsum32 = _make_segsum(H2 // 2, False, 8)


def _mm(a, b_t):
  return lax.dot_general(a, b_t, (((1,), (1,)), ((), ())),
                         preferred_element_type=jnp.float32)


def _tc1_body(x_ref, wl_ref, wr_ref, e_ref,
              qa_ref, qb_ref, r_ref, src_ref, dst_ref):
  xb = x_ref[...]
  q = _mm(xb, wl_ref[...])
  r_ref[...] = _mm(xb, wr_ref[...])
  qa_ref[...] = q[:, :H2]
  qb_ref[...] = q[:, H2:]
  i = pl.program_id(0)
  row = jax.lax.broadcasted_iota(jnp.int32, (EBLK, CH), 0) + i * EBLK
  valid = row < ER
  src_ref[...] = jnp.where(valid, e_ref[0], 0)
  dst_ref[...] = jnp.where(valid, e_ref[1], NN)


def _tc1(x, w1l, w1r, e3):
  return pl.pallas_call(
      _tc1_body,
      grid=(GRID,),
      in_specs=[
          pl.BlockSpec((BLK, DD), lambda i: (i, 0)),
          pl.BlockSpec((HH, DD), lambda i: (0, 0)),
          pl.BlockSpec((HH, DD), lambda i: (0, 0)),
          pl.BlockSpec((2, EBLK, CH), lambda i: (0, i, 0)),
      ],
      out_specs=[
          pl.BlockSpec((BLK, H2), lambda i: (i, 0)),
          pl.BlockSpec((BLK, H2), lambda i: (i, 0)),
          pl.BlockSpec((BLK, HH), lambda i: (i, 0)),
          pl.BlockSpec((EBLK, CH), lambda i: (i, 0)),
          pl.BlockSpec((EBLK, CH), lambda i: (i, 0)),
      ],
      out_shape=[
          jax.ShapeDtypeStruct((NN, H2), jnp.float32),
          jax.ShapeDtypeStruct((NN, H2), jnp.float32),
          jax.ShapeDtypeStruct((NN, HH), jnp.float32),
          jax.ShapeDtypeStruct((ERP, CH), jnp.int32),
          jax.ShapeDtypeStruct((ERP, CH), jnp.int32),
      ],
  )(x, w1l, w1r, e3)


def _tc2_body(agg_ref, cnt_ref, r1_ref, b1_ref, g1_ref, be1_ref,
              w2l_ref, w2r_ref, pa_ref, pb_ref, r2_ref, scl_ref):
  scale = 1.0 / jnp.maximum(cnt_ref[0, :, 0:1] + cnt_ref[1, :, 0:1], 1.0)
  agg = agg_ref[...] * scale
  h = agg + b1_ref[...] + r1_ref[...]
  h = h * (1.0 / jnp.sqrt(1.0 + BN_EPS)) * g1_ref[...] + be1_ref[...]
  h = jnp.maximum(h, 0.0)
  p = _mm(h, w2l_ref[...])
  pa_ref[...] = p[:, :H2 // 2]
  pb_ref[...] = p[:, H2 // 2:]
  r2_ref[...] = _mm(h, w2r_ref[...])
  scl_ref[...] = jnp.broadcast_to(scale, (BLK, 16))


def _tc2(agg1, cnt, r1, b1, g1, be1, w2l, w2r):
  return pl.pallas_call(
      _tc2_body,
      grid=(GRID,),
      in_specs=[
          pl.BlockSpec((BLK, 128), lambda i: (i, 0)),
          pl.BlockSpec((NSC, BLK, 16), lambda i: (0, i, 0)),
          pl.BlockSpec((BLK, HH), lambda i: (i, 0)),
          pl.BlockSpec((1, HH), lambda i: (0, 0)),
          pl.BlockSpec((1, HH), lambda i: (0, 0)),
          pl.BlockSpec((1, HH), lambda i: (0, 0)),
          pl.BlockSpec((H2, HH), lambda i: (0, 0)),
          pl.BlockSpec((H2, HH), lambda i: (0, 0)),
      ],
      out_specs=[
          pl.BlockSpec((BLK, H2 // 2), lambda i: (i, 0)),
          pl.BlockSpec((BLK, H2 // 2), lambda i: (i, 0)),
          pl.BlockSpec((BLK, H2), lambda i: (i, 0)),
          pl.BlockSpec((BLK, 16), lambda i: (i, 0)),
      ],
      out_shape=[
          jax.ShapeDtypeStruct((NN, H2 // 2), jnp.float32),
          jax.ShapeDtypeStruct((NN, H2 // 2), jnp.float32),
          jax.ShapeDtypeStruct((NN, H2), jnp.float32),
          jax.ShapeDtypeStruct((NN, 16), jnp.float32),
      ],
  )(agg1, cnt, r1, b1, g1, be1, w2l, w2r)


def _tc3_body(agg_ref, scl_ref, r2_ref, b2_ref, g2_ref, be2_ref,
              wh_ref, bh_ref, o_ref):
  scale = scl_ref[:, 0:1]
  agg = agg_ref[:, :H2] * scale
  h = agg + b2_ref[...] + r2_ref[...]
  h = h * (1.0 / jnp.sqrt(1.0 + BN_EPS)) * g2_ref[...] + be2_ref[...]
  h = jnp.maximum(h, 0.0)
  o_ref[...] = lax.dot_general(h, wh_ref[...], (((1,), (0,)), ((), ())),
                               preferred_element_type=jnp.float32) + bh_ref[...]


def _tc3(agg2, scl, r2, b2, g2, be2, whp, bhp):
  return pl.pallas_call(
      _tc3_body,
      grid=(GRID,),
      in_specs=[
          pl.BlockSpec((BLK, 128), lambda i: (i, 0)),
          pl.BlockSpec((BLK, 16), lambda i: (i, 0)),
          pl.BlockSpec((BLK, H2), lambda i: (i, 0)),
          pl.BlockSpec((1, H2), lambda i: (0, 0)),
          pl.BlockSpec((1, H2), lambda i: (0, 0)),
          pl.BlockSpec((1, H2), lambda i: (0, 0)),
          pl.BlockSpec((H2, 128), lambda i: (0, 0)),
          pl.BlockSpec((1, 128), lambda i: (0, 0)),
      ],
      out_specs=pl.BlockSpec((BLK, 128), lambda i: (i, 0)),
      out_shape=jax.ShapeDtypeStruct((NN, 128), jnp.float32),
  )(agg2, scl, r2, b2, g2, be2, whp, bhp)


def kernel(x, edge_index, W1_l, b1_l, W1_r, g1, be1,
           W2_l, b2_l, W2_r, g2, be2, Wh, bh):
  e3 = edge_index.reshape(2, ER, CH)

  zrow64 = jnp.zeros((RPT, H2), jnp.float32)
  zrow32 = jnp.zeros((RPT, H2 // 2), jnp.float32)
  zcnt = jnp.zeros((RPT, 16), jnp.float32)
  ones16 = jnp.zeros((CH, 16), jnp.float32).at[:, 0].set(1.0)

  qa, qb, r1, src_f, dst_f = _tc1(x, W1_l, W1_r, e3)
  src_r = src_f.reshape(NTILES, NCHUNK, CH)
  dst_r = dst_f.reshape(NTILES, NCHUNK, CH)
  agg1, cnt = _segsum64_cnt(qa, qb, src_r, dst_r, zrow64, zcnt, ones16)

  pa, pb, r2, scl = _tc2(agg1, cnt, r1, b1_l.reshape(1, HH), g1.reshape(1, HH),
                         be1.reshape(1, HH), W2_l, W2_r)
  agg2 = _segsum32(pa, pb, src_r, dst_r, zrow32)[0]

  whp = jnp.zeros((H2, 128), jnp.float32).at[:, :CC].set(Wh.T)
  bhp = jnp.zeros((1, 128), jnp.float32).at[0, :CC].set(bh)
  out = _tc3(agg2, scl, r2, b2_l.reshape(1, H2), g2.reshape(1, H2),
             be2.reshape(1, H2), whp, bhp)
  return out[:, :CC]
